# Initial kernel scaffold; baseline (speedup 1.0000x reference)
import jax
import jax.numpy as jnp
from jax.experimental import pallas as pl


def kernel(x, scaling_params, W, b, W1, b1, W2, b2, Wout, bout):
    raise NotImplementedError("write your pallas kernel here")


# speedup vs baseline: 20.7235x; 20.7235x over previous
import jax
import jax.numpy as jnp
from jax.experimental import pallas as pl
from jax.experimental.pallas import tpu as pltpu

H = 4
E = 8
K = 2
D_IN = 1024
FEAT = 1024
N = 8192
HID = 32 * H
BN = 1024

_NEG = -1e30


def _routing_kernel(sp_ref, idx_ref, probs_ref, reg_ref):
    sp = sp_ref[...]
    col = jax.lax.broadcasted_iota(jnp.int32, (H, E), 1)
    v0 = jnp.max(sp, axis=1, keepdims=True)
    i0 = jnp.min(jnp.where(sp >= v0, col, E), axis=1, keepdims=True)
    masked = jnp.where(col == i0, _NEG, sp)
    v1 = jnp.max(masked, axis=1, keepdims=True)
    i1 = jnp.min(jnp.where(masked >= v1, col, E), axis=1, keepdims=True)
    e1 = jnp.exp(v1 - v0)
    denom = 1.0 + e1
    p0 = 1.0 / denom
    p1 = e1 / denom
    idx_ref[...] = jnp.concatenate([i0, i1], axis=1)
    probs_ref[...] = jnp.concatenate([p0, p1], axis=1)
    P = jnp.where(col == i0, p0, 0.0) + jnp.where(col == i1, p1, 0.0)
    G = jnp.dot(P, P.T, preferred_element_type=jnp.float32)
    r = jax.lax.broadcasted_iota(jnp.int32, (H, H), 0)
    c = jax.lax.broadcasted_iota(jnp.int32, (H, H), 1)
    eye = jnp.where(r == c, 1.0, 0.0)
    reg_ref[...] = jnp.sum((G - eye) ** 2).reshape(1, 1)


def _mbuild_kernel(idx_ref, probs_ref, W_blk, W1_blk, b_blk, b1_blk,
                   M_ref, beff_ref):
    s = pl.program_id(0)
    h = s // K
    k = s % K
    p = probs_ref[h, k]

    @pl.when(s == 0)
    def _():
        M_ref[...] = jnp.zeros_like(M_ref)
        beff_ref[...] = b1_blk[...]

    Wm = W_blk[0, 0]
    W1m = W1_blk[0]
    bv = b_blk[0]
    M_ref[...] += p * jnp.dot(Wm, W1m, preferred_element_type=jnp.float32)
    beff_ref[...] += p * jnp.dot(bv, W1m, preferred_element_type=jnp.float32)


def _mlp_kernel(x_blk, M_blk, beff_blk, W2_blk, b2_blk, woutT_blk, bout_blk,
                out_ref):
    z1 = jnp.dot(x_blk[...], M_blk[...],
                 preferred_element_type=jnp.float32) + beff_blk[...]
    h1 = jax.nn.softplus(z1)
    z2 = jnp.dot(h1, W2_blk[...],
                 preferred_element_type=jnp.float32) + b2_blk[...]
    h2 = jax.nn.softplus(z2)
    out_ref[...] = (jnp.sum(h2 * woutT_blk[...], axis=1, keepdims=True)
                    + bout_blk[...])


def kernel(x, scaling_params, W, b, W1, b1, W2, b2, Wout, bout):
    f32 = jnp.float32

    idx, probs, reg = pl.pallas_call(
        _routing_kernel,
        out_shape=(
            jax.ShapeDtypeStruct((H, K), jnp.int32),
            jax.ShapeDtypeStruct((H, K), f32),
            jax.ShapeDtypeStruct((1, 1), f32),
        ),
    )(scaling_params)

    W1r = jnp.transpose(W1.reshape(FEAT, H, HID), (1, 0, 2))
    b_r = b.reshape(H * E, 1, FEAT)
    b1_r = b1.reshape(1, HID)

    grid_spec = pltpu.PrefetchScalarGridSpec(
        num_scalar_prefetch=2,
        grid=(H * K,),
        in_specs=[
            pl.BlockSpec((1, 1, D_IN, FEAT),
                         lambda s, idx_ref, pr_ref: (
                             s // K, idx_ref[s // K, s % K], 0, 0)),
            pl.BlockSpec((1, FEAT, HID),
                         lambda s, idx_ref, pr_ref: (s // K, 0, 0)),
            pl.BlockSpec((1, 1, FEAT),
                         lambda s, idx_ref, pr_ref: (
                             (s // K) * E + idx_ref[s // K, s % K], 0, 0)),
            pl.BlockSpec((1, HID), lambda s, idx_ref, pr_ref: (0, 0)),
        ],
        out_specs=[
            pl.BlockSpec((D_IN, HID), lambda s, idx_ref, pr_ref: (0, 0)),
            pl.BlockSpec((1, HID), lambda s, idx_ref, pr_ref: (0, 0)),
        ],
    )
    M, beff = pl.pallas_call(
        _mbuild_kernel,
        grid_spec=grid_spec,
        out_shape=(
            jax.ShapeDtypeStruct((D_IN, HID), f32),
            jax.ShapeDtypeStruct((1, HID), f32),
        ),
        compiler_params=pltpu.CompilerParams(
            dimension_semantics=("arbitrary",)),
    )(idx, probs, W, W1r, b_r, b1_r)

    out = pl.pallas_call(
        _mlp_kernel,
        grid=(N // BN,),
        in_specs=[
            pl.BlockSpec((BN, D_IN), lambda i: (i, 0)),
            pl.BlockSpec((D_IN, HID), lambda i: (0, 0)),
            pl.BlockSpec((1, HID), lambda i: (0, 0)),
            pl.BlockSpec((HID, HID), lambda i: (0, 0)),
            pl.BlockSpec((1, HID), lambda i: (0, 0)),
            pl.BlockSpec((1, HID), lambda i: (0, 0)),
            pl.BlockSpec((1, 1), lambda i: (0, 0)),
        ],
        out_specs=pl.BlockSpec((BN, 1), lambda i: (i, 0)),
        out_shape=jax.ShapeDtypeStruct((N, 1), f32),
        compiler_params=pltpu.CompilerParams(
            dimension_semantics=("parallel",)),
    )(x, M, beff, W2, b2.reshape(1, HID), Wout.reshape(1, HID),
      bout.reshape(1, 1))

    return out, reg[0, 0]
